# baseline (speedup 1.0000x reference)
import functools

import jax
import jax.numpy as jnp
from jax import lax
from jax.experimental import pallas as pl
from jax.experimental.pallas import tpu as pltpu
from jax.experimental.pallas import tpu_sc as plsc

B, S = 16384, 50
NUM_IDX = B * S
D = 32
NC = 2
NS = 16
NW = NC * NS
ROWS_PER_W = B // NW
CI = 32
CHUNK = CI * S
N_CHUNKS = ROWS_PER_W // CI
FH = D // 8
IH = B // 128
STAGE_BYTES = S * FH * 8 * CI * 4

_mesh = plsc.VectorSubcoreMesh(core_axis_name="c", subcore_axis_name="s")


@functools.partial(
    pl.kernel,
    mesh=_mesh,
    out_type=jax.ShapeDtypeStruct((S, FH, IH, 8, 128), jnp.float32),
    scratch_types=[
        pltpu.VMEM((CHUNK,), jnp.int32),
        pltpu.VMEM((CHUNK, D), jnp.float32),
        pltpu.VMEM((S, FH, 8, CI), jnp.float32),
        pltpu.SemaphoreType.DMA,
        pltpu.SemaphoreType.DMA,
    ],
    compiler_params=pltpu.CompilerParams(
        use_tc_tiling_on_sc=False, needs_layout_passes=False
    ),
)
def _embed_sc(idx_hbm, table_hbm, out_hbm, idx_v, rows_v, stage_v, gsem, wsem):
    wid = lax.axis_index("s") * NC + lax.axis_index("c")
    iota16 = lax.iota(jnp.int32, 16)
    row_lo = iota16 * S
    row_hi = row_lo + 16 * S

    def drain_stage():
        pltpu.make_async_copy(
            out_hbm.at[:, :, 0, :, pl.ds(0, CI)], stage_v, wsem
        ).wait()

    def body(c, _):
        i0 = wid * ROWS_PER_W + c * CI
        ih = 4 * wid + c // 4
        il0 = (c % 4) * CI
        pltpu.sync_copy(idx_hbm.at[pl.ds(i0 * S, CHUNK)], idx_v)
        pltpu.async_copy(table_hbm.at[idx_v], rows_v, gsem).wait()

        @pl.when(c > 0)
        def _():
            drain_stage()

        def transpose_j(j, _):
            r0 = row_lo + j
            r1 = row_hi + j
            for f in range(D):
                col = jnp.full((16,), f, jnp.int32)
                v0 = plsc.load_gather(rows_v, [r0, col])
                v1 = plsc.load_gather(rows_v, [r1, col])
                stage_v[j, f // 8, f % 8, pl.ds(0, 16)] = v0
                stage_v[j, f // 8, f % 8, pl.ds(16, 16)] = v1
            return ()

        lax.fori_loop(0, S, transpose_j, ())

        def wb_j(j, _):
            for fh in range(FH):
                pltpu.async_copy(
                    stage_v.at[j, fh],
                    out_hbm.at[j, fh, ih, :, pl.ds(il0, CI)],
                    wsem,
                )
            return ()

        lax.fori_loop(0, S, wb_j, ())
        return ()

    lax.fori_loop(0, N_CHUNKS, body, ())
    drain_stage()


def kernel(x, embedding):
    idx = x.astype(jnp.int32).reshape(NUM_IDX)
    out5 = _embed_sc(idx, embedding)
    return jnp.transpose(out5, (2, 4, 0, 1, 3)).reshape(B, S, D)


# speedup vs baseline: 1.5726x; 1.5726x over previous
import functools

import jax
import jax.numpy as jnp
from jax import lax
from jax.experimental import pallas as pl
from jax.experimental.pallas import tpu as pltpu
from jax.experimental.pallas import tpu_sc as plsc

B, S = 16384, 50
NUM_IDX = B * S
D = 32
NC = 2
NS = 16
NW = NC * NS
ROWS_PER_W = B // NW
CI = 32
CHUNK = CI * S
N_CHUNKS = ROWS_PER_W // CI
FH = D // 8
IH = B // 128
STAGE_BYTES = S * FH * 8 * CI * 4

_mesh = plsc.VectorSubcoreMesh(core_axis_name="c", subcore_axis_name="s")


@functools.partial(
    pl.kernel,
    mesh=_mesh,
    out_type=jax.ShapeDtypeStruct((S, FH, IH, 8, 128), jnp.float32),
    scratch_types=[
        pltpu.VMEM((CHUNK,), jnp.int32),
        pltpu.VMEM((CHUNK, D), jnp.float32),
        pltpu.VMEM((S, FH, 8, CI), jnp.float32),
        pltpu.SemaphoreType.DMA,
        pltpu.SemaphoreType.DMA,
    ],
    compiler_params=pltpu.CompilerParams(
        use_tc_tiling_on_sc=False, needs_layout_passes=False
    ),
)
def _embed_sc(idx_hbm, table_hbm, out_hbm, idx_v, rows_v, stage_v, gsem, wsem):
    wid = lax.axis_index("s") * NC + lax.axis_index("c")
    iota16 = lax.iota(jnp.int32, 16)
    row_lo = iota16 * S
    row_hi = row_lo + 16 * S

    def drain_stage():
        pltpu.make_async_copy(
            out_hbm.at[:, :, 0, :, pl.ds(0, CI)], stage_v, wsem
        ).wait()

    def body(c, _):
        i0 = wid * ROWS_PER_W + c * CI
        ih = 4 * wid + c // 4
        il0 = (c % 4) * CI
        pltpu.sync_copy(idx_hbm.at[pl.ds(i0 * S, CHUNK)], idx_v)
        pltpu.async_copy(table_hbm.at[idx_v], rows_v, gsem).wait()

        @pl.when(c > 0)
        def _():
            drain_stage()

        for f in range(D):
            rot = (iota16 + f) & (D - 1)
            fh_i = rot >> 3
            fl_i = rot & 7

            def transpose_j(j, _, rot=rot, fh_i=fh_i, fl_i=fl_i):
                jv = jnp.full((16,), 0, jnp.int32) + j
                v0 = plsc.load_gather(rows_v, [row_lo + j, rot])
                v1 = plsc.load_gather(rows_v, [row_hi + j, rot])
                plsc.store_scatter(stage_v, [jv, fh_i, fl_i, iota16], v0)
                plsc.store_scatter(stage_v, [jv, fh_i, fl_i, iota16 + 16], v1)
                return ()

            lax.fori_loop(0, S, transpose_j, ())

        def wb_j(j, _):
            for fh in range(FH):
                pltpu.async_copy(
                    stage_v.at[j, fh],
                    out_hbm.at[j, fh, ih, :, pl.ds(il0, CI)],
                    wsem,
                )
            return ()

        lax.fori_loop(0, S, wb_j, ())
        return ()

    lax.fori_loop(0, N_CHUNKS, body, ())
    drain_stage()


def kernel(x, embedding):
    idx = x.astype(jnp.int32).reshape(NUM_IDX)
    out5 = _embed_sc(idx, embedding)
    return jnp.transpose(out5, (2, 4, 0, 1, 3)).reshape(B, S, D)
